# baseline (speedup 1.0000x reference)
import functools

import jax
import jax.numpy as jnp
from jax.experimental import pallas as pl
from jax.experimental.pallas import tpu as pltpu


def _greedy_search_kernel(L_hbm, W_ref, sos_ref, c_ref, q_ref,
                          LC_ref, LT_ref, NT_ref, sems,
                          *, B, C, T_l, J):
    W = W_ref[:]

    chunks = [(k * 128, min((k + 1) * 128, C)) for k in range((C + 127) // 128)]
    copies = [
        pltpu.make_async_copy(L_hbm.at[lo:hi], LC_ref.at[lo:hi], sems.at[k])
        for k, (lo, hi) in enumerate(chunks)
    ]
    for cp in copies:
        cp.start()
    for k, (lo, hi) in enumerate(chunks):
        copies[k].wait()
        for s in range(T_l):
            blk = jnp.transpose(LC_ref[lo:hi, s, :], (1, 0))
            LT_ref[s, :, lo:hi] = blk
            NT_ref[s, lo:hi] = jnp.sum(blk * blk, axis=0)

    nt_pre = [jnp.zeros((1, C), jnp.float32)]
    for s in range(T_l):
        nt_pre.append(nt_pre[-1] + NT_ref[s:s + 1, :])

    lane_iota = jax.lax.broadcasted_iota(jnp.int32, (1, C), 1)

    def argmin_row(sim):
        m = jnp.min(sim)
        return jnp.min(jnp.where(sim == m, lane_iota, C))

    q0 = jnp.tanh(jnp.dot(sos_ref[:], W, preferred_element_type=jnp.float32))
    d0 = nt_pre[1] - 2.0 * jnp.sum(LT_ref[0] * q0.reshape(J, 1), axis=0,
                                   keepdims=True)
    c = argmin_row(d0)

    for t in range(1, T_l + 1):
        chosen = LC_ref[pl.ds(c, 1), :, :].reshape(T_l, J)
        q = jnp.tanh(jnp.dot(chosen, W, preferred_element_type=jnp.float32))
        if t == T_l:
            q_ref[:] = jnp.broadcast_to(q[None], (B, T_l, J))
        cross = jnp.sum(jnp.sum(LT_ref[0:t] * q[0:t, :, None], axis=1),
                        axis=0, keepdims=True)
        sim = nt_pre[t] - 2.0 * cross
        c = argmin_row(sim)

    c_ref[:] = jnp.full((B,), c, dtype=jnp.int32)


def kernel(x, lens, W, label_seqs, sos):
    B = x.shape[0]
    C, T_l, J = label_seqs.shape

    n_chunks = (C + 127) // 128
    pred_label_sofar, pred_label_seq = pl.pallas_call(
        functools.partial(_greedy_search_kernel, B=B, C=C, T_l=T_l, J=J),
        in_specs=[
            pl.BlockSpec(memory_space=pltpu.MemorySpace.HBM),
            pl.BlockSpec(memory_space=pltpu.MemorySpace.VMEM),
            pl.BlockSpec(memory_space=pltpu.MemorySpace.VMEM),
        ],
        out_shape=(
            jax.ShapeDtypeStruct((B,), jnp.int32),
            jax.ShapeDtypeStruct((B, T_l, J), jnp.float32),
        ),
        scratch_shapes=[
            pltpu.VMEM((C, T_l, J), jnp.float32),
            pltpu.VMEM((T_l, J, C), jnp.float32),
            pltpu.VMEM((T_l, C), jnp.float32),
            pltpu.SemaphoreType.DMA((n_chunks,)),
        ],
    )(label_seqs, W, sos.reshape(1, J))

    return (pred_label_sofar, pred_label_seq)


# speedup vs baseline: 1.1007x; 1.1007x over previous
import functools

import jax
import jax.numpy as jnp
from jax.experimental import pallas as pl
from jax.experimental.pallas import tpu as pltpu


def _greedy_search_kernel(L_ref, W_ref, sos_ref, c_ref, q_ref, LT_ref,
                          *, B, C, T_l, J):
    W = W_ref[:]

    nt_pre = [jnp.zeros((1, C), jnp.float32)]

    def relayout_row(s):
        lt_s = jnp.transpose(L_ref[:, s, :], (1, 0))
        LT_ref[s] = lt_s
        nt_pre.append(nt_pre[-1] + jnp.sum(lt_s * lt_s, axis=0, keepdims=True))

    lane_iota = jax.lax.broadcasted_iota(jnp.int32, (1, C), 1)

    def argmin_row(sim):
        m = jnp.min(sim)
        return jnp.min(jnp.where(sim == m, lane_iota, C))

    relayout_row(0)
    q0 = jnp.tanh(jnp.dot(sos_ref[:], W, preferred_element_type=jnp.float32))
    d0 = nt_pre[1] - 2.0 * jnp.sum(LT_ref[0] * q0.reshape(J, 1), axis=0,
                                   keepdims=True)
    c = argmin_row(d0)

    for t in range(1, T_l + 1):
        if t < T_l:
            relayout_row(t)
        chosen = L_ref[pl.ds(c, 1), :, :].reshape(T_l, J)
        q = jnp.tanh(jnp.dot(chosen, W, preferred_element_type=jnp.float32))
        if t == T_l:
            q_ref[:] = jnp.broadcast_to(q[None], (B, T_l, J))
        cross = jnp.sum(jnp.sum(LT_ref[0:t] * q[0:t, :, None], axis=1),
                        axis=0, keepdims=True)
        sim = nt_pre[t] - 2.0 * cross
        c = argmin_row(sim)

    c_ref[:] = jnp.full((B,), c, dtype=jnp.int32)


def kernel(x, lens, W, label_seqs, sos):
    B = x.shape[0]
    C, T_l, J = label_seqs.shape

    pred_label_sofar, pred_label_seq = pl.pallas_call(
        functools.partial(_greedy_search_kernel, B=B, C=C, T_l=T_l, J=J),
        out_shape=(
            jax.ShapeDtypeStruct((B,), jnp.int32),
            jax.ShapeDtypeStruct((B, T_l, J), jnp.float32),
        ),
        scratch_shapes=[pltpu.VMEM((T_l, J, C), jnp.float32)],
    )(label_seqs, W, sos.reshape(1, J))

    return (pred_label_sofar, pred_label_seq)


# speedup vs baseline: 1.3149x; 1.1946x over previous
import functools

import jax
import jax.numpy as jnp
from jax.experimental import pallas as pl
from jax.experimental.pallas import tpu as pltpu


def _greedy_search_kernel(L_ref, W_ref, sos_ref, c_ref, q_ref, LT_ref,
                          *, B, C, T_l, J):
    W = W_ref[:]

    nt_pre = [jnp.zeros((1, C), jnp.float32)]

    def relayout_row(s):
        lt_s = jnp.transpose(L_ref[:, s, :], (1, 0))
        LT_ref[s] = lt_s
        nt_pre.append(nt_pre[-1] + jnp.sum(lt_s * lt_s, axis=0, keepdims=True))

    lane_iota = jax.lax.broadcasted_iota(jnp.int32, (1, C), 1)

    def argmin_row(sim):
        m = jnp.min(sim)
        return jnp.min(jnp.where(sim == m, lane_iota, C))

    relayout_row(0)
    q0 = jnp.tanh(jnp.dot(sos_ref[:], W, preferred_element_type=jnp.float32))
    d0 = nt_pre[1] - 2.0 * jnp.sum(LT_ref[0] * q0.reshape(J, 1), axis=0,
                                   keepdims=True)
    c = argmin_row(d0)

    for t in range(1, T_l + 1):
        if t < T_l:
            relayout_row(t)
        chosen = L_ref[pl.ds(c, 1), :, :].reshape(T_l, J)
        q = jnp.tanh(jnp.dot(chosen, W, preferred_element_type=jnp.float32))
        if t == T_l:
            q_ref[:] = jnp.broadcast_to(q[None], (B, T_l, J))
        m = (t // 3) * 2 if t >= 3 else 0
        cross = jnp.zeros((1, C), jnp.float32)
        if m > 0:
            qflat = q[0:m].reshape(1, m * J)
            ltstack = LT_ref[0:m].reshape(m * J, C)
            cross = cross + jnp.dot(qflat, ltstack,
                                    preferred_element_type=jnp.float32)
        if m < t:
            cross = cross + jnp.sum(
                jnp.sum(LT_ref[m:t] * q[m:t, :, None], axis=1),
                axis=0, keepdims=True)
        sim = nt_pre[t] - 2.0 * cross
        c = argmin_row(sim)

    c_ref[:] = jnp.full((B,), c, dtype=jnp.int32)


def kernel(x, lens, W, label_seqs, sos):
    B = x.shape[0]
    C, T_l, J = label_seqs.shape

    pred_label_sofar, pred_label_seq = pl.pallas_call(
        functools.partial(_greedy_search_kernel, B=B, C=C, T_l=T_l, J=J),
        out_shape=(
            jax.ShapeDtypeStruct((B,), jnp.int32),
            jax.ShapeDtypeStruct((B, T_l, J), jnp.float32),
        ),
        scratch_shapes=[pltpu.VMEM((T_l, J, C), jnp.float32)],
    )(label_seqs, W, sos.reshape(1, J))

    return (pred_label_sofar, pred_label_seq)
